# baseline (speedup 1.0000x reference)
import functools

import jax
import jax.numpy as jnp
from jax import lax
from jax.experimental import pallas as pl
from jax.experimental.pallas import tpu as pltpu
from jax.experimental.pallas import tpu_sc as plsc

_N_STIMULI = 30
_N_DIM = 10
_BATCH = 16384
_BETA = 3.0
_MIDPOINT = 0.5
_RATE = 5.0

_LANES = 16
_NUM_WORKERS = 32
_BPW = _BATCH // _NUM_WORKERS
_TROWS = 32
_TCOLS = 16
_NCONST = _TROWS * _TCOLS + _N_DIM * _LANES


@functools.partial(
    pl.kernel,
    mesh=plsc.VectorSubcoreMesh(core_axis_name="c", subcore_axis_name="s"),
    compiler_params=pltpu.CompilerParams(needs_layout_passes=False),
    out_type=jax.ShapeDtypeStruct((_BATCH,), jnp.float32),
    scratch_types=[
        pltpu.VMEM((_NCONST,), jnp.float32),
        pltpu.VMEM((_BPW,), jnp.int32),
        pltpu.VMEM((_BPW,), jnp.int32),
        pltpu.VMEM((_BPW,), jnp.float32),
        pltpu.SemaphoreType.DMA,
        pltpu.SemaphoreType.DMA,
        pltpu.SemaphoreType.DMA,
    ],
)
def _rate_sim_sc(const_hbm, i_hbm, j_hbm, out_hbm,
                 const_ref, i_ref, j_ref, o_ref, sem0, sem1, sem2):
    nc = 2
    wid = lax.axis_index("s") * nc + lax.axis_index("c")
    base = wid * _BPW

    cp0 = pltpu.async_copy(const_hbm, const_ref, sem0)
    cp1 = pltpu.async_copy(i_hbm.at[pl.ds(base, _BPW)], i_ref, sem1)
    cp2 = pltpu.async_copy(j_hbm.at[pl.ds(base, _BPW)], j_ref, sem2)
    cp0.wait()
    cp1.wait()
    cp2.wait()

    wbase = _TROWS * _TCOLS
    wvecs = [
        const_ref[pl.ds(wbase + d * _LANES, _LANES)] for d in range(_N_DIM)
    ]

    @plsc.parallel_loop(0, _BPW // _LANES, unroll=4)
    def _chunk(c):
        off = c * _LANES
        iv = i_ref[pl.ds(off, _LANES)]
        jv = j_ref[pl.ds(off, _LANES)]
        ia = iv * _TCOLS
        ja = jv * _TCOLS
        acc = jnp.zeros((_LANES,), jnp.float32)
        for d in range(_N_DIM):
            za = plsc.load_gather(const_ref, [ia + d])
            zb = plsc.load_gather(const_ref, [ja + d])
            df = za - zb
            acc = acc + wvecs[d] * df * df
        acc = jnp.maximum(acc, jnp.float32(1e-30))
        bits = lax.bitcast_convert_type(acc, jnp.int32)
        y = lax.bitcast_convert_type(
            jnp.int32(0x5F3759DF) - (bits >> 1), jnp.float32)
        for _ in range(3):
            y = y * (1.5 - 0.5 * acc * y * y)
        dist = acc * y
        s = jnp.exp(-_BETA * dist)
        prob = 1.0 / (1.0 + jnp.exp(_RATE * _MIDPOINT - _RATE * s))
        o_ref[pl.ds(off, _LANES)] = prob

    pltpu.sync_copy(o_ref, out_hbm.at[pl.ds(base, _BPW)])


def kernel(inputs, table, w):
    i_arr = jnp.asarray(inputs[:, 0], jnp.int32)
    j_arr = jnp.asarray(inputs[:, 1], jnp.int32)
    tab = jnp.zeros((_TROWS, _TCOLS), jnp.float32)
    tab = tab.at[: _N_STIMULI + 1, : _N_DIM].set(table)
    wb = jnp.broadcast_to(
        w.astype(jnp.float32)[:, None], (_N_DIM, _LANES))
    consts = jnp.concatenate([tab.reshape(-1), wb.reshape(-1)])
    return _rate_sim_sc(consts, i_arr, j_arr)


# speedup vs baseline: 1.0336x; 1.0336x over previous
import functools

import jax
import jax.numpy as jnp
from jax import lax
from jax.experimental import pallas as pl
from jax.experimental.pallas import tpu as pltpu
from jax.experimental.pallas import tpu_sc as plsc

_N_STIMULI = 30
_N_DIM = 10
_BATCH = 16384
_BETA = 3.0
_MIDPOINT = 0.5
_RATE = 5.0

_LANES = 16
_NUM_WORKERS = 32
_BPW = _BATCH // _NUM_WORKERS
_TROWS = 32
_TCOLS = 16
_NCONST = _TROWS * _TCOLS + _N_DIM * _LANES


@functools.partial(
    pl.kernel,
    mesh=plsc.VectorSubcoreMesh(core_axis_name="c", subcore_axis_name="s"),
    compiler_params=pltpu.CompilerParams(needs_layout_passes=False),
    out_type=jax.ShapeDtypeStruct((_BATCH,), jnp.float32),
    scratch_types=[
        pltpu.VMEM((_NCONST,), jnp.float32),
        pltpu.VMEM((2 * _BPW,), jnp.float32),
        pltpu.VMEM((_BPW,), jnp.float32),
        pltpu.SemaphoreType.DMA,
        pltpu.SemaphoreType.DMA,
    ],
)
def _rate_sim_sc(packed_hbm, out_hbm, const_ref, ij_ref, o_ref, sem0, sem1):
    nc = 2
    wid = lax.axis_index("s") * nc + lax.axis_index("c")
    base = wid * _BPW

    cp0 = pltpu.async_copy(packed_hbm.at[pl.ds(0, _NCONST)], const_ref, sem0)
    cp1 = pltpu.async_copy(
        packed_hbm.at[pl.ds(_NCONST + 2 * base, 2 * _BPW)], ij_ref, sem1)
    cp0.wait()
    cp1.wait()

    wbase = _TROWS * _TCOLS
    wvecs = [
        const_ref[pl.ds(wbase + d * _LANES, _LANES)] for d in range(_N_DIM)
    ]

    @plsc.parallel_loop(0, _BPW // _LANES, unroll=2)
    def _chunk(c):
        off = c * _LANES
        iv = plsc.bitcast(ij_ref[pl.ds(off, _LANES)], jnp.int32)
        jv = plsc.bitcast(ij_ref[pl.ds(_BPW + off, _LANES)], jnp.int32)
        ia = iv * _TCOLS
        ja = jv * _TCOLS
        acc = jnp.zeros((_LANES,), jnp.float32)
        for d in range(_N_DIM):
            za = plsc.load_gather(const_ref, [ia + d])
            zb = plsc.load_gather(const_ref, [ja + d])
            df = za - zb
            acc = acc + wvecs[d] * df * df
        acc = jnp.maximum(acc, jnp.float32(1e-30))
        bits = lax.bitcast_convert_type(acc, jnp.int32)
        y = lax.bitcast_convert_type(
            jnp.int32(0x5F3759DF) - (bits >> 1), jnp.float32)
        for _ in range(3):
            y = y * (1.5 - 0.5 * acc * y * y)
        dist = acc * y
        s = jnp.exp(-_BETA * dist)
        prob = 1.0 / (1.0 + jnp.exp(_RATE * _MIDPOINT - _RATE * s))
        o_ref[pl.ds(off, _LANES)] = prob

    pltpu.sync_copy(o_ref, out_hbm.at[pl.ds(base, _BPW)])


def kernel(inputs, table, w):
    tab = jnp.zeros((_TROWS, _TCOLS), jnp.float32)
    tab = tab.at[: _N_STIMULI + 1, : _N_DIM].set(table)
    wb = jnp.broadcast_to(w.astype(jnp.float32)[:, None], (_N_DIM, _LANES))
    ij = jnp.stack(
        [
            inputs[:, 0].astype(jnp.int32).reshape(_NUM_WORKERS, _BPW),
            inputs[:, 1].astype(jnp.int32).reshape(_NUM_WORKERS, _BPW),
        ],
        axis=1,
    )
    packed = jnp.concatenate([
        tab.reshape(-1),
        wb.reshape(-1),
        lax.bitcast_convert_type(ij.reshape(-1), jnp.float32),
    ])
    return _rate_sim_sc(packed)


# speedup vs baseline: 1.0381x; 1.0043x over previous
import functools

import jax
import jax.numpy as jnp
from jax import lax
from jax.experimental import pallas as pl
from jax.experimental.pallas import tpu as pltpu
from jax.experimental.pallas import tpu_sc as plsc

_N_STIMULI = 30
_N_DIM = 10
_BATCH = 16384
_BETA = 3.0
_MIDPOINT = 0.5
_RATE = 5.0

_LANES = 16
_NUM_WORKERS = 32
_BPW = _BATCH // _NUM_WORKERS
_TROWS = 32
_TCOLS = 16
_NCONST = _TROWS * _TCOLS + _N_DIM * _LANES


@functools.partial(
    pl.kernel,
    mesh=plsc.VectorSubcoreMesh(core_axis_name="c", subcore_axis_name="s"),
    compiler_params=pltpu.CompilerParams(needs_layout_passes=False),
    out_type=jax.ShapeDtypeStruct((_BATCH,), jnp.float32),
    scratch_types=[
        pltpu.VMEM((_NCONST,), jnp.float32),
        pltpu.VMEM((_BPW,), jnp.float32),
        pltpu.VMEM((_BPW,), jnp.float32),
        pltpu.SemaphoreType.DMA,
        pltpu.SemaphoreType.DMA,
    ],
)
def _rate_sim_sc(packed_hbm, out_hbm, const_ref, ij_ref, o_ref, sem0, sem1):
    nc = 2
    wid = lax.axis_index("s") * nc + lax.axis_index("c")
    base = wid * _BPW

    cp0 = pltpu.async_copy(packed_hbm.at[pl.ds(0, _NCONST)], const_ref, sem0)
    cp1 = pltpu.async_copy(
        packed_hbm.at[pl.ds(_NCONST + base, _BPW)], ij_ref, sem1)
    cp0.wait()
    cp1.wait()

    wbase = _TROWS * _TCOLS
    wvecs = [
        const_ref[pl.ds(wbase + d * _LANES, _LANES)] for d in range(_N_DIM)
    ]

    @plsc.parallel_loop(0, _BPW // _LANES, unroll=2)
    def _chunk(c):
        off = c * _LANES
        pv = plsc.bitcast(ij_ref[pl.ds(off, _LANES)], jnp.int32)
        ia = (pv >> 5) * _TCOLS
        ja = (pv & 31) * _TCOLS
        acc = jnp.zeros((_LANES,), jnp.float32)
        for d in range(_N_DIM):
            za = plsc.load_gather(const_ref, [ia + d])
            zb = plsc.load_gather(const_ref, [ja + d])
            df = za - zb
            acc = acc + wvecs[d] * df * df
        acc = jnp.maximum(acc, jnp.float32(1e-30))
        bits = lax.bitcast_convert_type(acc, jnp.int32)
        y = lax.bitcast_convert_type(
            jnp.int32(0x5F3759DF) - (bits >> 1), jnp.float32)
        for _ in range(3):
            y = y * (1.5 - 0.5 * acc * y * y)
        dist = acc * y
        s = jnp.exp(-_BETA * dist)
        prob = 1.0 / (1.0 + jnp.exp(_RATE * _MIDPOINT - _RATE * s))
        o_ref[pl.ds(off, _LANES)] = prob

    pltpu.sync_copy(o_ref, out_hbm.at[pl.ds(base, _BPW)])


def kernel(inputs, table, w):
    tab = jnp.zeros((_TROWS, _TCOLS), jnp.float32)
    tab = tab.at[: _N_STIMULI + 1, : _N_DIM].set(table)
    wb = jnp.broadcast_to(w.astype(jnp.float32)[:, None], (_N_DIM, _LANES))
    pij = inputs[:, 0].astype(jnp.int32) * 32 + inputs[:, 1].astype(jnp.int32)
    packed = jnp.concatenate([
        tab.reshape(-1),
        wb.reshape(-1),
        lax.bitcast_convert_type(pij, jnp.float32),
    ])
    return _rate_sim_sc(packed)
